# baseline (speedup 1.0000x reference)
import jax
import jax.numpy as jnp
from jax.experimental import pallas as pl

TEXT_DIM = 512
NUM_CLASSES = 20
KPAD = 32
OUT_H = 128
OUT_W = 256
IN_W = 512
GROUP = 8
C_TILE = 128


def _emb_kernel(seg_ref, lhs_ref, sel_ref, out_ref):
    rows = seg_ref[0, :, 0:IN_W].astype(jnp.float32)
    rowds = jax.lax.dot_general(
        rows, sel_ref[...], (((1,), (0,)), ((), ())),
        preferred_element_type=jnp.float32)
    rid = rowds.astype(jnp.int32)
    kio = jax.lax.broadcasted_iota(jnp.int32, (KPAD * GROUP, OUT_W), 0)
    krep = jnp.bitwise_and(kio, KPAD - 1)
    lhs = lhs_ref[...]
    for g in range(OUT_H // GROUP):
        rrep = jnp.concatenate(
            [jnp.broadcast_to(rid[g * GROUP + r:g * GROUP + r + 1, :],
                              (KPAD, OUT_W))
             for r in range(GROUP)], axis=0)
        oh = (krep == rrep).astype(jnp.float32)
        res = jax.lax.dot_general(
            lhs, oh, (((1,), (0,)), ((), ())),
            preferred_element_type=jnp.float32)
        out_ref[0, :, g * GROUP:(g + 1) * GROUP, :] = res.reshape(
            C_TILE, GROUP, OUT_W)


@jax.jit
def kernel(seg_map, class_embeddings):
    bs = seg_map.shape[0]
    seg_r = seg_map.reshape(bs, OUT_H, 2 * IN_W)
    etp = jnp.zeros((TEXT_DIM, KPAD), jnp.float32)
    etp = etp.at[:, :NUM_CLASSES].set(class_embeddings.T)
    eye = jnp.eye(GROUP, dtype=jnp.float32)
    lhs = (eye[None, :, :, None] * etp[:, None, None, :]).reshape(
        TEXT_DIM * GROUP, GROUP * KPAD)
    sel = (jax.lax.broadcasted_iota(jnp.int32, (IN_W, OUT_W), 0)
           == 2 * jax.lax.broadcasted_iota(jnp.int32, (IN_W, OUT_W), 1)
           ).astype(jnp.float32)
    grid = (bs, TEXT_DIM // C_TILE)
    return pl.pallas_call(
        _emb_kernel,
        grid=grid,
        in_specs=[
            pl.BlockSpec((1, OUT_H, 2 * IN_W), lambda b, c: (b, 0, 0)),
            pl.BlockSpec((C_TILE * GROUP, GROUP * KPAD),
                         lambda b, c: (c, 0)),
            pl.BlockSpec((IN_W, OUT_W), lambda b, c: (0, 0)),
        ],
        out_specs=pl.BlockSpec(
            (1, C_TILE, OUT_H, OUT_W), lambda b, c: (b, c, 0, 0)),
        out_shape=jax.ShapeDtypeStruct(
            (bs, TEXT_DIM, OUT_H, OUT_W), jnp.float32),
    )(seg_r, lhs, sel)


# speedup vs baseline: 1.0647x; 1.0647x over previous
import jax
import jax.numpy as jnp
from jax.experimental import pallas as pl

TEXT_DIM = 512
NUM_CLASSES = 20
KPAD = 32
OUT_H = 128
OUT_W = 256
IN_W = 512
GROUP = 8
C_TILE = 128


def _emb_kernel(seg_ref, lhs_ref, sel_ref, out_ref):
    rows = seg_ref[0, :, 0:IN_W].astype(jnp.bfloat16)
    rowds = jax.lax.dot_general(
        rows, sel_ref[...], (((1,), (0,)), ((), ())),
        preferred_element_type=jnp.float32)
    rid = rowds.astype(jnp.int32)
    kio = jax.lax.broadcasted_iota(jnp.int32, (KPAD * GROUP, OUT_W), 0)
    krep = jnp.bitwise_and(kio, KPAD - 1)
    lhs = lhs_ref[...]
    for g in range(OUT_H // GROUP):
        rrep = jnp.concatenate(
            [jnp.broadcast_to(rid[g * GROUP + r:g * GROUP + r + 1, :],
                              (KPAD, OUT_W))
             for r in range(GROUP)], axis=0)
        oh = (krep == rrep).astype(jnp.bfloat16)
        res = jax.lax.dot_general(
            lhs, oh, (((1,), (0,)), ((), ())),
            preferred_element_type=jnp.float32)
        out_ref[0, :, g * GROUP:(g + 1) * GROUP, :] = res.reshape(
            C_TILE, GROUP, OUT_W)


@jax.jit
def kernel(seg_map, class_embeddings):
    bs = seg_map.shape[0]
    seg_r = seg_map.reshape(bs, OUT_H, 2 * IN_W)
    etp = jnp.zeros((TEXT_DIM, KPAD), jnp.float32)
    etp = etp.at[:, :NUM_CLASSES].set(class_embeddings.T)
    eye = jnp.eye(GROUP, dtype=jnp.float32)
    lhs = (eye[None, :, :, None] * etp[:, None, None, :]).reshape(
        TEXT_DIM * GROUP, GROUP * KPAD).astype(jnp.bfloat16)
    sel = (jax.lax.broadcasted_iota(jnp.int32, (IN_W, OUT_W), 0)
           == 2 * jax.lax.broadcasted_iota(jnp.int32, (IN_W, OUT_W), 1)
           ).astype(jnp.bfloat16)
    grid = (bs, TEXT_DIM // C_TILE)
    return pl.pallas_call(
        _emb_kernel,
        grid=grid,
        in_specs=[
            pl.BlockSpec((1, OUT_H, 2 * IN_W), lambda b, c: (b, 0, 0)),
            pl.BlockSpec((C_TILE * GROUP, GROUP * KPAD),
                         lambda b, c: (c, 0)),
            pl.BlockSpec((IN_W, OUT_W), lambda b, c: (0, 0)),
        ],
        out_specs=pl.BlockSpec(
            (1, C_TILE, OUT_H, OUT_W), lambda b, c: (b, c, 0, 0)),
        out_shape=jax.ShapeDtypeStruct(
            (bs, TEXT_DIM, OUT_H, OUT_W), jnp.float32),
    )(seg_r, lhs, sel)


# speedup vs baseline: 1.0894x; 1.0233x over previous
import jax
import jax.numpy as jnp
from jax.experimental import pallas as pl

TEXT_DIM = 512
NUM_CLASSES = 20
KPAD = 32
OUT_H = 128
OUT_W = 256
IN_W = 512
GROUP = 8
C_TILE = 64


def _emb_kernel(seg_ref, lhs_ref, sel_ref, out_ref):
    rows = seg_ref[0, :, 0:IN_W].astype(jnp.bfloat16)
    rowds = jax.lax.dot_general(
        rows, sel_ref[...], (((1,), (0,)), ((), ())),
        preferred_element_type=jnp.float32)
    rid = rowds.astype(jnp.int32)
    kio = jax.lax.broadcasted_iota(jnp.int32, (KPAD * GROUP, OUT_W), 0)
    krep = jnp.bitwise_and(kio, KPAD - 1)
    lhs = lhs_ref[...]
    for g in range(OUT_H // GROUP):
        rrep = jnp.concatenate(
            [jnp.broadcast_to(rid[g * GROUP + r:g * GROUP + r + 1, :],
                              (KPAD, OUT_W))
             for r in range(GROUP)], axis=0)
        oh = (krep == rrep).astype(jnp.bfloat16)
        res = jax.lax.dot_general(
            lhs, oh, (((1,), (0,)), ((), ())),
            preferred_element_type=jnp.float32)
        out_ref[0, :, g * GROUP:(g + 1) * GROUP, :] = res.reshape(
            C_TILE, GROUP, OUT_W)


@jax.jit
def kernel(seg_map, class_embeddings):
    bs = seg_map.shape[0]
    seg_r = seg_map.reshape(bs, OUT_H, 2 * IN_W)
    etp = jnp.zeros((TEXT_DIM, KPAD), jnp.float32)
    etp = etp.at[:, :NUM_CLASSES].set(class_embeddings.T)
    eye = jnp.eye(GROUP, dtype=jnp.float32)
    lhs = (eye[None, :, :, None] * etp[:, None, None, :]).reshape(
        TEXT_DIM * GROUP, GROUP * KPAD).astype(jnp.bfloat16)
    sel = (jax.lax.broadcasted_iota(jnp.int32, (IN_W, OUT_W), 0)
           == 2 * jax.lax.broadcasted_iota(jnp.int32, (IN_W, OUT_W), 1)
           ).astype(jnp.bfloat16)
    grid = (bs, TEXT_DIM // C_TILE)
    return pl.pallas_call(
        _emb_kernel,
        grid=grid,
        in_specs=[
            pl.BlockSpec((1, OUT_H, 2 * IN_W), lambda b, c: (b, 0, 0)),
            pl.BlockSpec((C_TILE * GROUP, GROUP * KPAD),
                         lambda b, c: (c, 0)),
            pl.BlockSpec((IN_W, OUT_W), lambda b, c: (0, 0)),
        ],
        out_specs=pl.BlockSpec(
            (1, C_TILE, OUT_H, OUT_W), lambda b, c: (b, c, 0, 0)),
        out_shape=jax.ShapeDtypeStruct(
            (bs, TEXT_DIM, OUT_H, OUT_W), jnp.float32),
    )(seg_r, lhs, sel)
